# baseline (speedup 1.0000x reference)
import functools

import jax
import jax.numpy as jnp
from jax import lax
from jax.experimental import pallas as pl
from jax.experimental.pallas import tpu as pltpu
from jax.experimental.pallas import tpu_sc as plsc

N = 10000
NP = 10240
E = 320000
EP = 327680
D = 128
HA = 80
HP = 64
NC, NS = 2, 16
NW = NC * NS
RPS = NP // NS
RB = 1024
_PREC = lax.Precision.HIGHEST

_mesh = plsc.VectorSubcoreMesh(core_axis_name="c", subcore_axis_name="s")


def _make_propagate(w, ring, eb, rows0, rows1):
    assert 16 * (rows0 + rows1) * eb == EP
    assert rows0 % ring == 0 and rows1 % ring == 0

    @functools.partial(
        pl.kernel,
        out_type=jax.ShapeDtypeStruct((NC, 2, NP, w), jnp.float32),
        mesh=_mesh,
        scratch_types=[
            pltpu.VMEM((2, ring, eb), jnp.int32),
            pltpu.VMEM((2, ring, eb), jnp.int32),
            pltpu.VMEM((ring * eb, w), jnp.float32),
            pltpu.VMEM_SHARED((NP, w), jnp.float32),
            pltpu.VMEM_SHARED((NP, w), jnp.float32),
        ] + [pltpu.SemaphoreType.DMA] * (2 * ring + 2),
        compiler_params=pltpu.CompilerParams(use_tc_tiling_on_sc=False),
    )
    def _propagate(tbl, src2, dst2, zrows, out, sidx, didx, rows, tbl_sh,
                   agg_sh, *sems):
        gsem = sems[:ring]
        ssem = sems[ring:2 * ring]
        isem = sems[2 * ring:]
        cid = lax.axis_index("c")
        sid = lax.axis_index("s")
        rows_c = rows0 + cid * (rows1 - rows0)
        n_rounds = rows_c // ring
        row0 = cid * (NS * rows0) + sid * rows_c
        my_nodes = pl.ds(sid * RPS, RPS)

        def buf(b):
            return rows.at[pl.ds(b * eb, eb)]

        def dummy_wait(dst, sem):
            pltpu.make_async_copy(tbl.at[0, pl.ds(0, dst.shape[0])], dst,
                                  sem).wait()

        for h in range(2):
            pltpu.sync_copy(tbl.at[h, my_nodes], tbl_sh.at[my_nodes])
            pltpu.sync_copy(zrows, agg_sh.at[my_nodes])
            pltpu.sync_copy(src2.at[pl.ds(row0, ring)], sidx.at[0])
            pltpu.sync_copy(dst2.at[pl.ds(row0, ring)], didx.at[0])
            plsc.subcore_barrier()

            @pl.loop(0, n_rounds)
            def _round(it):
                p = lax.rem(it, 2)

                @pl.when(it + 1 < n_rounds)
                def _():
                    nr = row0 + (it + 1) * ring
                    pltpu.async_copy(src2.at[pl.ds(nr, ring)],
                                     sidx.at[1 - p], isem[0])
                    pltpu.async_copy(dst2.at[pl.ds(nr, ring)],
                                     didx.at[1 - p], isem[1])

                @pl.when(it > 0)
                def _():
                    pltpu.make_async_copy(src2.at[pl.ds(0, ring)],
                                          sidx.at[p], isem[0]).wait()
                    pltpu.make_async_copy(dst2.at[pl.ds(0, ring)],
                                          didx.at[p], isem[1]).wait()

                for b in range(ring):
                    pltpu.async_copy(tbl_sh.at[sidx.at[p, b]], buf(b),
                                     gsem[b])
                for b in range(ring):
                    dummy_wait(buf(b), gsem[b])
                    pltpu.async_copy(buf(b), agg_sh.at[didx.at[p, b]],
                                     ssem[b], add=True)
                for b in range(ring):
                    dummy_wait(buf(b), ssem[b])

            plsc.subcore_barrier()
            pltpu.sync_copy(agg_sh.at[my_nodes], out.at[cid, h, my_nodes])

    return _propagate


_prop_aug = _make_propagate(HA, 8, 32, 272, 368)
_prop = _make_propagate(HP, 8, 32, 272, 368)


def _tc_layer0_body(a_ref, w_ref, b_ref, o_ref, dinv_ref):
    aL = a_ref[0, 0] + a_ref[1, 0]
    aR = a_ref[0, 1] + a_ref[1, 1]
    deg = aR[:, 48:49]
    dinv = 1.0 / jnp.maximum(deg, 1.0)
    y = jnp.dot(aL * dinv, w_ref[:HA, :], preferred_element_type=jnp.float32,
                precision=_PREC)
    y = y + jnp.dot(aR[:, :48] * dinv, w_ref[HA:D, :],
                    preferred_element_type=jnp.float32, precision=_PREC)
    y = jnp.maximum(y + b_ref[...], 0.0)
    o_ref[0] = y[:, :HP]
    o_ref[1] = y[:, HP:]
    dinv_ref[...] = dinv


_tc_layer0 = pl.pallas_call(
    _tc_layer0_body,
    grid=(NP // RB,),
    in_specs=[
        pl.BlockSpec((NC, 2, RB, HA), lambda i: (0, 0, i, 0)),
        pl.BlockSpec((D, D), lambda i: (0, 0)),
        pl.BlockSpec((1, D), lambda i: (0, 0)),
    ],
    out_specs=[
        pl.BlockSpec((2, RB, HP), lambda i: (0, i, 0)),
        pl.BlockSpec((RB, 1), lambda i: (i, 0)),
    ],
    out_shape=[
        jax.ShapeDtypeStruct((2, NP, HP), jnp.float32),
        jax.ShapeDtypeStruct((NP, 1), jnp.float32),
    ],
)


def _tc_layer_body(a_ref, dinv_ref, w_ref, b_ref, o_ref, *, relu):
    dinv = dinv_ref[...]
    aL = (a_ref[0, 0] + a_ref[1, 0]) * dinv
    aR = (a_ref[0, 1] + a_ref[1, 1]) * dinv
    y = jnp.dot(aL, w_ref[:HP, :], preferred_element_type=jnp.float32,
                precision=_PREC)
    y = y + jnp.dot(aR, w_ref[HP:D, :], preferred_element_type=jnp.float32,
                    precision=_PREC)
    y = y + b_ref[...]
    if relu:
        y = jnp.maximum(y, 0.0)
        o_ref[0] = y[:, :HP]
        o_ref[1] = y[:, HP:]
    else:
        o_ref[...] = y


def _make_tc_layer(relu):
    if relu:
        out_specs = pl.BlockSpec((2, RB, HP), lambda i: (0, i, 0))
        out_shape = jax.ShapeDtypeStruct((2, NP, HP), jnp.float32)
    else:
        out_specs = pl.BlockSpec((RB, D), lambda i: (i, 0))
        out_shape = jax.ShapeDtypeStruct((NP, D), jnp.float32)
    return pl.pallas_call(
        functools.partial(_tc_layer_body, relu=relu),
        grid=(NP // RB,),
        in_specs=[
            pl.BlockSpec((NC, 2, RB, HP), lambda i: (0, 0, i, 0)),
            pl.BlockSpec((RB, 1), lambda i: (i, 0)),
            pl.BlockSpec((D, D), lambda i: (0, 0)),
            pl.BlockSpec((1, D), lambda i: (0, 0)),
        ],
        out_specs=out_specs,
        out_shape=out_shape,
    )


_tc_layer_relu = _make_tc_layer(True)
_tc_layer_lin = _make_tc_layer(False)


def kernel(x, edge_index, W0, b0, W1, b1, W2, b2):
    src = edge_index[0].astype(jnp.int32)
    dst = edge_index[1].astype(jnp.int32)
    srcp = jnp.concatenate([src, jnp.zeros((EP - E,), jnp.int32)])
    dstp = jnp.concatenate([dst, jnp.full((EP - E,), N, jnp.int32)])
    src32, dst32 = srcp.reshape(EP // 32, 32), dstp.reshape(EP // 32, 32)
    xa0 = jnp.zeros((NP, HA), jnp.float32).at[:N].set(x[:, :HA])
    xa1 = jnp.zeros((NP, HA), jnp.float32)
    xa1 = xa1.at[:N, :48].set(x[:, HA:]).at[:N, 48].set(1.0)
    xa = jnp.stack([xa0, xa1])
    zrows_a = jnp.zeros((RPS, HA), jnp.float32)
    zrows = jnp.zeros((RPS, HP), jnp.float32)

    W2p = jnp.zeros((D, D), jnp.float32).at[:, :2].set(W2)
    b2p = jnp.zeros((D,), jnp.float32).at[:2].set(b2)

    a1 = _prop_aug(xa, src32, dst32, zrows_a)
    h, dinv = _tc_layer0(a1, W0, b0.reshape(1, D))
    a = _prop(h, src32, dst32, zrows)
    h = _tc_layer_relu(a, dinv, W1, b1.reshape(1, D))
    a = _prop(h, src32, dst32, zrows)
    out = _tc_layer_lin(a, dinv, W2p, b2p.reshape(1, D))
    return out[:N, :2]


# speedup vs baseline: 1.1228x; 1.1228x over previous
import functools

import jax
import jax.numpy as jnp
from jax import lax
from jax.experimental import pallas as pl
from jax.experimental.pallas import tpu as pltpu
from jax.experimental.pallas import tpu_sc as plsc

N = 10000
NP = 10240
E = 320000
EP = 327680
D = 128
HA = 80
HP = 64
NC, NS = 2, 16
NW = NC * NS
RPS = NP // NS
RB = 1024
_PREC = lax.Precision.HIGHEST

_mesh = plsc.VectorSubcoreMesh(core_axis_name="c", subcore_axis_name="s")


def _make_propagate(w, ring, eb, rows0, rows1):
    assert 16 * (rows0 + rows1) * eb == EP
    assert rows0 % ring == 0 and rows1 % ring == 0

    @functools.partial(
        pl.kernel,
        out_type=jax.ShapeDtypeStruct((NC, 2, NP, w), jnp.float32),
        mesh=_mesh,
        scratch_types=[
            pltpu.VMEM((2, ring, eb), jnp.int32),
            pltpu.VMEM((2, ring, eb), jnp.int32),
            pltpu.VMEM((ring * eb, w), jnp.float32),
            pltpu.VMEM_SHARED((NP, w), jnp.float32),
            pltpu.VMEM_SHARED((NP, w), jnp.float32),
        ] + [pltpu.SemaphoreType.DMA] * (2 * ring + 2),
        compiler_params=pltpu.CompilerParams(use_tc_tiling_on_sc=False),
    )
    def _propagate(tbl, src2, dst2, zrows, out, sidx, didx, rows, tbl_sh,
                   agg_sh, *sems):
        gsem = sems[:ring]
        ssem = sems[ring:2 * ring]
        isem = sems[2 * ring:]
        cid = lax.axis_index("c")
        sid = lax.axis_index("s")
        rows_c = rows0 + cid * (rows1 - rows0)
        n_rounds = rows_c // ring
        row0 = cid * (NS * rows0) + sid * rows_c
        my_nodes = pl.ds(sid * RPS, RPS)

        def buf(b):
            return rows.at[pl.ds(b * eb, eb)]

        def dummy_wait(dst, sem):
            pltpu.make_async_copy(tbl.at[0, pl.ds(0, dst.shape[0])], dst,
                                  sem).wait()

        for h in range(2):
            pltpu.sync_copy(tbl.at[h, my_nodes], tbl_sh.at[my_nodes])
            pltpu.sync_copy(zrows, agg_sh.at[my_nodes])
            pltpu.sync_copy(src2.at[pl.ds(row0, ring)], sidx.at[0])
            pltpu.sync_copy(dst2.at[pl.ds(row0, ring)], didx.at[0])
            plsc.subcore_barrier()

            @pl.loop(0, n_rounds)
            def _round(it):
                p = lax.rem(it, 2)

                @pl.when(it + 1 < n_rounds)
                def _():
                    nr = row0 + (it + 1) * ring
                    pltpu.async_copy(src2.at[pl.ds(nr, ring)],
                                     sidx.at[1 - p], isem[0])
                    pltpu.async_copy(dst2.at[pl.ds(nr, ring)],
                                     didx.at[1 - p], isem[1])

                @pl.when(it > 0)
                def _():
                    pltpu.make_async_copy(src2.at[pl.ds(0, ring)],
                                          sidx.at[p], isem[0]).wait()
                    pltpu.make_async_copy(dst2.at[pl.ds(0, ring)],
                                          didx.at[p], isem[1]).wait()

                for b in range(ring):
                    pltpu.async_copy(tbl_sh.at[sidx.at[p, b]], buf(b),
                                     gsem[b])
                for b in range(ring):
                    dummy_wait(buf(b), gsem[b])
                    pltpu.async_copy(buf(b), agg_sh.at[didx.at[p, b]],
                                     ssem[b], add=True)
                for b in range(ring):
                    dummy_wait(buf(b), ssem[b])

            plsc.subcore_barrier()
            pltpu.sync_copy(agg_sh.at[my_nodes], out.at[cid, h, my_nodes])

    return _propagate


_prop_aug = _make_propagate(HA, 8, 32, 352, 288)
_prop = _make_propagate(HP, 8, 32, 352, 288)


def _tc_layer0_body(a_ref, w_ref, b_ref, o_ref, dinv_ref):
    aL = a_ref[0, 0] + a_ref[1, 0]
    aR = a_ref[0, 1] + a_ref[1, 1]
    deg = aR[:, 48:49]
    dinv = 1.0 / jnp.maximum(deg, 1.0)
    y = jnp.dot(aL * dinv, w_ref[:HA, :], preferred_element_type=jnp.float32,
                precision=_PREC)
    y = y + jnp.dot(aR[:, :48] * dinv, w_ref[HA:D, :],
                    preferred_element_type=jnp.float32, precision=_PREC)
    y = jnp.maximum(y + b_ref[...], 0.0)
    o_ref[0] = y[:, :HP]
    o_ref[1] = y[:, HP:]
    dinv_ref[...] = dinv


_tc_layer0 = pl.pallas_call(
    _tc_layer0_body,
    grid=(NP // RB,),
    in_specs=[
        pl.BlockSpec((NC, 2, RB, HA), lambda i: (0, 0, i, 0)),
        pl.BlockSpec((D, D), lambda i: (0, 0)),
        pl.BlockSpec((1, D), lambda i: (0, 0)),
    ],
    out_specs=[
        pl.BlockSpec((2, RB, HP), lambda i: (0, i, 0)),
        pl.BlockSpec((RB, 1), lambda i: (i, 0)),
    ],
    out_shape=[
        jax.ShapeDtypeStruct((2, NP, HP), jnp.float32),
        jax.ShapeDtypeStruct((NP, 1), jnp.float32),
    ],
)


def _tc_layer_body(a_ref, dinv_ref, w_ref, b_ref, o_ref, *, relu):
    dinv = dinv_ref[...]
    aL = (a_ref[0, 0] + a_ref[1, 0]) * dinv
    aR = (a_ref[0, 1] + a_ref[1, 1]) * dinv
    y = jnp.dot(aL, w_ref[:HP, :], preferred_element_type=jnp.float32,
                precision=_PREC)
    y = y + jnp.dot(aR, w_ref[HP:D, :], preferred_element_type=jnp.float32,
                    precision=_PREC)
    y = y + b_ref[...]
    if relu:
        y = jnp.maximum(y, 0.0)
        o_ref[0] = y[:, :HP]
        o_ref[1] = y[:, HP:]
    else:
        o_ref[...] = y


def _make_tc_layer(relu):
    if relu:
        out_specs = pl.BlockSpec((2, RB, HP), lambda i: (0, i, 0))
        out_shape = jax.ShapeDtypeStruct((2, NP, HP), jnp.float32)
    else:
        out_specs = pl.BlockSpec((RB, D), lambda i: (i, 0))
        out_shape = jax.ShapeDtypeStruct((NP, D), jnp.float32)
    return pl.pallas_call(
        functools.partial(_tc_layer_body, relu=relu),
        grid=(NP // RB,),
        in_specs=[
            pl.BlockSpec((NC, 2, RB, HP), lambda i: (0, 0, i, 0)),
            pl.BlockSpec((RB, 1), lambda i: (i, 0)),
            pl.BlockSpec((D, D), lambda i: (0, 0)),
            pl.BlockSpec((1, D), lambda i: (0, 0)),
        ],
        out_specs=out_specs,
        out_shape=out_shape,
    )


_tc_layer_relu = _make_tc_layer(True)
_tc_layer_lin = _make_tc_layer(False)


def kernel(x, edge_index, W0, b0, W1, b1, W2, b2):
    src = edge_index[0].astype(jnp.int32)
    dst = edge_index[1].astype(jnp.int32)
    srcp = jnp.concatenate([src, jnp.zeros((EP - E,), jnp.int32)])
    dstp = jnp.concatenate([dst, jnp.full((EP - E,), N, jnp.int32)])
    src32, dst32 = srcp.reshape(EP // 32, 32), dstp.reshape(EP // 32, 32)
    xa0 = jnp.zeros((NP, HA), jnp.float32).at[:N].set(x[:, :HA])
    xa1 = jnp.zeros((NP, HA), jnp.float32)
    xa1 = xa1.at[:N, :48].set(x[:, HA:]).at[:N, 48].set(1.0)
    xa = jnp.stack([xa0, xa1])
    zrows_a = jnp.zeros((RPS, HA), jnp.float32)
    zrows = jnp.zeros((RPS, HP), jnp.float32)

    W2p = jnp.zeros((D, D), jnp.float32).at[:, :2].set(W2)
    b2p = jnp.zeros((D,), jnp.float32).at[:2].set(b2)

    a1 = _prop_aug(xa, src32, dst32, zrows_a)
    h, dinv = _tc_layer0(a1, W0, b0.reshape(1, D))
    a = _prop(h, src32, dst32, zrows)
    h = _tc_layer_relu(a, dinv, W1, b1.reshape(1, D))
    a = _prop(h, src32, dst32, zrows)
    out = _tc_layer_lin(a, dinv, W2p, b2p.reshape(1, D))
    return out[:N, :2]


# speedup vs baseline: 1.1470x; 1.0216x over previous
import functools

import jax
import jax.numpy as jnp
from jax import lax
from jax.experimental import pallas as pl
from jax.experimental.pallas import tpu as pltpu
from jax.experimental.pallas import tpu_sc as plsc

N = 10000
NP = 10240
E = 320000
EP = 327680
D = 128
HA = 80
HP = 64
NC, NS = 2, 16
NW = NC * NS
RPS = NP // NS
RB = 1024
_PREC = lax.Precision.HIGHEST

_mesh = plsc.VectorSubcoreMesh(core_axis_name="c", subcore_axis_name="s")


def _make_propagate(w, ring, eb, rows0, rows1):
    assert 16 * (rows0 + rows1) * eb == EP
    assert rows0 % ring == 0 and rows1 % ring == 0

    @functools.partial(
        pl.kernel,
        out_type=jax.ShapeDtypeStruct((NC, 2, NP, w), jnp.float32),
        mesh=_mesh,
        scratch_types=[
            pltpu.VMEM((2, ring, eb), jnp.int32),
            pltpu.VMEM((2, ring, eb), jnp.int32),
            pltpu.VMEM((ring * eb, w), jnp.float32),
            pltpu.VMEM_SHARED((NP, w), jnp.float32),
            pltpu.VMEM_SHARED((NP, w), jnp.float32),
        ] + [pltpu.SemaphoreType.DMA] * (2 * ring + 2),
        compiler_params=pltpu.CompilerParams(use_tc_tiling_on_sc=False),
    )
    def _propagate(tbl, src2, dst2, zrows, out, sidx, didx, rows, tbl_sh,
                   agg_sh, *sems):
        gsem = sems[:ring]
        ssem = sems[ring:2 * ring]
        isem = sems[2 * ring:]
        cid = lax.axis_index("c")
        sid = lax.axis_index("s")
        rows_c = rows0 + cid * (rows1 - rows0)
        n_rounds = rows_c // ring
        row0 = cid * (NS * rows0) + sid * rows_c
        my_nodes = pl.ds(sid * RPS, RPS)

        def buf(b):
            return rows.at[pl.ds(b * eb, eb)]

        def dummy_wait(dst, sem):
            pltpu.make_async_copy(tbl.at[0, pl.ds(0, dst.shape[0])], dst,
                                  sem).wait()

        for h in range(2):
            pltpu.sync_copy(tbl.at[h, my_nodes], tbl_sh.at[my_nodes])
            pltpu.sync_copy(zrows, agg_sh.at[my_nodes])
            pltpu.sync_copy(src2.at[pl.ds(row0, ring)], sidx.at[0])
            pltpu.sync_copy(dst2.at[pl.ds(row0, ring)], didx.at[0])
            plsc.subcore_barrier()

            @pl.loop(0, n_rounds)
            def _round(it):
                p = lax.rem(it, 2)

                @pl.when(it + 1 < n_rounds)
                def _():
                    nr = row0 + (it + 1) * ring
                    pltpu.async_copy(src2.at[pl.ds(nr, ring)],
                                     sidx.at[1 - p], isem[0])
                    pltpu.async_copy(dst2.at[pl.ds(nr, ring)],
                                     didx.at[1 - p], isem[1])

                @pl.when(it > 0)
                def _():
                    pltpu.make_async_copy(src2.at[pl.ds(0, ring)],
                                          sidx.at[p], isem[0]).wait()
                    pltpu.make_async_copy(dst2.at[pl.ds(0, ring)],
                                          didx.at[p], isem[1]).wait()

                for b in range(ring):
                    pltpu.async_copy(tbl_sh.at[sidx.at[p, b]], buf(b),
                                     gsem[b])
                for b in range(ring):
                    dummy_wait(buf(b), gsem[b])
                    pltpu.async_copy(buf(b), agg_sh.at[didx.at[p, b]],
                                     ssem[b], add=True)
                for b in range(ring):
                    dummy_wait(buf(b), ssem[b])

            plsc.subcore_barrier()
            pltpu.sync_copy(agg_sh.at[my_nodes], out.at[cid, h, my_nodes])

    return _propagate


_prop_aug = _make_propagate(HA, 8, 32, 384, 256)
_prop = _make_propagate(HP, 8, 32, 384, 256)


def _tc_layer0_body(a_ref, w_ref, b_ref, o_ref, dinv_ref):
    aL = a_ref[0, 0] + a_ref[1, 0]
    aR = a_ref[0, 1] + a_ref[1, 1]
    deg = aR[:, 48:49]
    dinv = 1.0 / jnp.maximum(deg, 1.0)
    y = jnp.dot(aL * dinv, w_ref[:HA, :], preferred_element_type=jnp.float32,
                precision=_PREC)
    y = y + jnp.dot(aR[:, :48] * dinv, w_ref[HA:D, :],
                    preferred_element_type=jnp.float32, precision=_PREC)
    y = jnp.maximum(y + b_ref[...], 0.0)
    o_ref[0] = y[:, :HP]
    o_ref[1] = y[:, HP:]
    dinv_ref[...] = dinv


_tc_layer0 = pl.pallas_call(
    _tc_layer0_body,
    grid=(NP // RB,),
    in_specs=[
        pl.BlockSpec((NC, 2, RB, HA), lambda i: (0, 0, i, 0)),
        pl.BlockSpec((D, D), lambda i: (0, 0)),
        pl.BlockSpec((1, D), lambda i: (0, 0)),
    ],
    out_specs=[
        pl.BlockSpec((2, RB, HP), lambda i: (0, i, 0)),
        pl.BlockSpec((RB, 1), lambda i: (i, 0)),
    ],
    out_shape=[
        jax.ShapeDtypeStruct((2, NP, HP), jnp.float32),
        jax.ShapeDtypeStruct((NP, 1), jnp.float32),
    ],
)


def _tc_layer_body(a_ref, dinv_ref, w_ref, b_ref, o_ref, *, relu):
    dinv = dinv_ref[...]
    aL = (a_ref[0, 0] + a_ref[1, 0]) * dinv
    aR = (a_ref[0, 1] + a_ref[1, 1]) * dinv
    y = jnp.dot(aL, w_ref[:HP, :], preferred_element_type=jnp.float32,
                precision=_PREC)
    y = y + jnp.dot(aR, w_ref[HP:D, :], preferred_element_type=jnp.float32,
                    precision=_PREC)
    y = y + b_ref[...]
    if relu:
        y = jnp.maximum(y, 0.0)
        o_ref[0] = y[:, :HP]
        o_ref[1] = y[:, HP:]
    else:
        o_ref[...] = y


def _make_tc_layer(relu):
    if relu:
        out_specs = pl.BlockSpec((2, RB, HP), lambda i: (0, i, 0))
        out_shape = jax.ShapeDtypeStruct((2, NP, HP), jnp.float32)
    else:
        out_specs = pl.BlockSpec((RB, D), lambda i: (i, 0))
        out_shape = jax.ShapeDtypeStruct((NP, D), jnp.float32)
    return pl.pallas_call(
        functools.partial(_tc_layer_body, relu=relu),
        grid=(NP // RB,),
        in_specs=[
            pl.BlockSpec((NC, 2, RB, HP), lambda i: (0, 0, i, 0)),
            pl.BlockSpec((RB, 1), lambda i: (i, 0)),
            pl.BlockSpec((D, D), lambda i: (0, 0)),
            pl.BlockSpec((1, D), lambda i: (0, 0)),
        ],
        out_specs=out_specs,
        out_shape=out_shape,
    )


_tc_layer_relu = _make_tc_layer(True)
_tc_layer_lin = _make_tc_layer(False)


def kernel(x, edge_index, W0, b0, W1, b1, W2, b2):
    src = edge_index[0].astype(jnp.int32)
    dst = edge_index[1].astype(jnp.int32)
    srcp = jnp.concatenate([src, jnp.zeros((EP - E,), jnp.int32)])
    dstp = jnp.concatenate([dst, jnp.full((EP - E,), N, jnp.int32)])
    src32, dst32 = srcp.reshape(EP // 32, 32), dstp.reshape(EP // 32, 32)
    xa0 = jnp.zeros((NP, HA), jnp.float32).at[:N].set(x[:, :HA])
    xa1 = jnp.zeros((NP, HA), jnp.float32)
    xa1 = xa1.at[:N, :48].set(x[:, HA:]).at[:N, 48].set(1.0)
    xa = jnp.stack([xa0, xa1])
    zrows_a = jnp.zeros((RPS, HA), jnp.float32)
    zrows = jnp.zeros((RPS, HP), jnp.float32)

    W2p = jnp.zeros((D, D), jnp.float32).at[:, :2].set(W2)
    b2p = jnp.zeros((D,), jnp.float32).at[:2].set(b2)

    a1 = _prop_aug(xa, src32, dst32, zrows_a)
    h, dinv = _tc_layer0(a1, W0, b0.reshape(1, D))
    a = _prop(h, src32, dst32, zrows)
    h = _tc_layer_relu(a, dinv, W1, b1.reshape(1, D))
    a = _prop(h, src32, dst32, zrows)
    out = _tc_layer_lin(a, dinv, W2p, b2p.reshape(1, D))
    return out[:N, :2]
